# Initial kernel scaffold; baseline (speedup 1.0000x reference)
import jax
import jax.numpy as jnp
from jax.experimental import pallas as pl


def kernel(pred, target):
    raise NotImplementedError("write your pallas kernel here")


# speedup vs baseline: 2.5674x; 2.5674x over previous
import functools

import jax
import jax.numpy as jnp
from jax.experimental import pallas as pl
from jax.experimental.pallas import tpu as pltpu

_SMOOTH = 0.1


def _loss_body(t_ref, x_ref, out_ref, m_ref, s_ref, sum_ref, p_ref,
               *, nj, cb, c, rb, nrows):
    i = pl.program_id(0)
    j = pl.program_id(1)

    @pl.when(j == 0)
    def _init_row_state():
        m_ref[...] = jnp.full((rb, 1), -jnp.inf, dtype=jnp.float32)
        s_ref[...] = jnp.zeros((rb, 1), dtype=jnp.float32)
        sum_ref[...] = jnp.zeros((rb, 1), dtype=jnp.float32)
        p_ref[...] = jnp.zeros((rb, 1), dtype=jnp.float32)

    @pl.when((i == 0) & (j == 0))
    def _init_out():
        out_ref[0, 0] = 0.0

    x = x_ref[...]
    t_loc = t_ref[0] - j * cb
    col = jax.lax.broadcasted_iota(jnp.int32, (rb, cb), 1)

    def _update(xv, xs):
        tmask = col == t_loc
        p_ref[...] += jnp.sum(jnp.where(tmask, xs, 0.0), axis=1, keepdims=True)
        sum_ref[...] += jnp.sum(xs, axis=1, keepdims=True)
        m_old = m_ref[...]
        m_new = jnp.maximum(m_old, jnp.max(xv, axis=1, keepdims=True))
        e = jnp.exp(xv - m_new)
        s_ref[...] = (s_ref[...] * jnp.exp(m_old - m_new)
                      + jnp.sum(e, axis=1, keepdims=True))
        m_ref[...] = m_new

    @pl.when(j < nj - 1)
    def _full_block():
        _update(x, x)

    @pl.when(j == nj - 1)
    def _tail_block():
        valid = col < (c - (nj - 1) * cb)
        _update(jnp.where(valid, x, -jnp.inf), jnp.where(valid, x, 0.0))

        a = _SMOOTH / (c - 1)
        b = (1.0 - _SMOOTH) - a
        lse = m_ref[...] + jnp.log(s_ref[...])
        row_loss = a * (c * lse - sum_ref[...]) + b * (lse - p_ref[...])
        out_ref[0, 0] += jnp.sum(row_loss) / nrows


def kernel(pred, target):
    nrows, c = pred.shape
    rb = 256 if nrows % 256 == 0 else nrows
    cb = min(8192, ((c + 127) // 128) * 128)
    ni = nrows // rb
    nj = (c + cb - 1) // cb

    t3 = target.astype(jnp.int32).reshape(ni, rb, 1)

    out = pl.pallas_call(
        functools.partial(_loss_body, nj=nj, cb=cb, c=c, rb=rb, nrows=nrows),
        grid=(ni, nj),
        in_specs=[
            pl.BlockSpec((1, rb, 1), lambda i, j: (i, 0, 0)),
            pl.BlockSpec((rb, cb), lambda i, j: (i, j)),
        ],
        out_specs=pl.BlockSpec(memory_space=pltpu.SMEM),
        out_shape=jax.ShapeDtypeStruct((1, 1), jnp.float32),
        scratch_shapes=[
            pltpu.VMEM((rb, 1), jnp.float32),
            pltpu.VMEM((rb, 1), jnp.float32),
            pltpu.VMEM((rb, 1), jnp.float32),
            pltpu.VMEM((rb, 1), jnp.float32),
        ],
        compiler_params=pltpu.CompilerParams(
            dimension_semantics=("arbitrary", "arbitrary"),
        ),
    )(t3, pred)
    return out.reshape(())
